# baseline (speedup 1.0000x reference)
import functools

import jax
import jax.numpy as jnp
import numpy as np
from jax import lax
from jax.experimental import pallas as pl
from jax.experimental.pallas import tpu as pltpu
from jax.experimental.pallas import tpu_sc as plsc

N = 10000
E = 320000
NC, NS = 2, 16
NW = NC * NS
PT = E // NW
C = 80
CHUNKS = PT // C
RPT = N // NS

_EYE8 = np.eye(8, dtype=np.float32)
_K8 = np.kron(_EYE8, np.ones((1, 8), np.float32))
_K4 = np.kron(_EYE8, np.ones((1, 4), np.float32))
_SELN = np.concatenate([np.eye(8), np.zeros((4, 8))], 0)
_PN = np.kron(_EYE8, _SELN.T.astype(np.float32))
_SELE = np.concatenate([np.zeros((8, 4)), np.eye(4)], 0)
_PE = np.kron(_EYE8, _SELE.T.astype(np.float32))
_M8 = np.kron(_EYE8, np.ones((8, 1), np.float32))


def _f16(v):
    return jnp.full((16,), v, dtype=jnp.float32)


def _i16(v):
    return jnp.full((16,), v, dtype=jnp.int32)


def _sc_edge_pass1(table, adst, edge_index, ea, cvec, zeros):
    H, D, CA, WT, WA = 8, 8, 64, 80, 80
    mesh = plsc.VectorSubcoreMesh(core_axis_name="c", subcore_axis_name="s",
                                  num_cores=NC, num_subcores=NS)

    @functools.partial(
        pl.kernel,
        out_type=jax.ShapeDtypeStruct((NC, N, WA), jnp.float32),
        mesh=mesh,
        compiler_params=pltpu.CompilerParams(use_tc_tiling_on_sc=False,
                                             needs_layout_passes=False),
        scratch_types=[
            pltpu.VMEM((PT,), jnp.int32),
            pltpu.VMEM((PT,), jnp.int32),
            [pltpu.VMEM((C, WT), jnp.float32)] * 2,
            [pltpu.VMEM((C, 16), jnp.float32)] * 2,
            [pltpu.VMEM((C, 1), jnp.float32)] * 2,
            [pltpu.VMEM((C, WA), jnp.float32)] * 2,
            pltpu.VMEM((16, 16), jnp.float32),
            pltpu.VMEM_SHARED((N, WA), jnp.float32),
            [pltpu.SemaphoreType.DMA] * 2,
            [pltpu.SemaphoreType.DMA] * 2,
        ],
    )
    def body(table_r, adst_r, edge_r, ea_r, cvec_r, zeros_r, out_r,
             srcv, dstv, recb, drecb, eab, outb, cb, accum, sg, ss):
        ci = lax.axis_index("c")
        si = lax.axis_index("s")
        wid = si * NC + ci
        g0 = wid * PT
        iota = lax.iota(jnp.int32, 16)

        pltpu.sync_copy(edge_r.at[0].at[pl.ds(g0, PT)], srcv)
        pltpu.sync_copy(edge_r.at[1].at[pl.ds(g0, PT)], dstv)
        pltpu.sync_copy(cvec_r, cb)
        pltpu.sync_copy(zeros_r.at[pl.ds(si * RPT, RPT)],
                        accum.at[pl.ds(si * RPT, RPT)])

        ch = [plsc.load_gather(cb, [_i16(h), iota]) for h in range(H)]
        plsc.subcore_barrier()

        def issue_gathers(i, k):
            pltpu.async_copy(table_r.at[srcv.at[pl.ds(i * C, C)]],
                             recb[k], sg[k])
            pltpu.async_copy(adst_r.at[dstv.at[pl.ds(i * C, C)]],
                             drecb[k], sg[k])
            pltpu.async_copy(ea_r.at[pl.ds(g0 + i * C, C)], eab[k], sg[k])

        def wait_gathers(i, k):
            pltpu.make_async_copy(table_r.at[srcv.at[pl.ds(i * C, C)]],
                                  recb[k], sg[k]).wait()
            pltpu.make_async_copy(adst_r.at[dstv.at[pl.ds(i * C, C)]],
                                  drecb[k], sg[k]).wait()
            pltpu.make_async_copy(ea_r.at[pl.ds(g0 + i * C, C)],
                                  eab[k], sg[k]).wait()

        def issue_scatter(i, k):
            pltpu.async_copy(outb[k], accum.at[dstv.at[pl.ds(i * C, C)]],
                             ss[k], add=True)

        def wait_scatter(i, k):
            pltpu.make_async_copy(outb[k], accum.at[dstv.at[pl.ds(i * C, C)]],
                                  ss[k]).wait()

        def compute(i, k):
            ibase = _i16(0) + i * C

            def grp(jb):
                rows = jb + iota
                ea16 = plsc.load_gather(eab[k], [rows, _i16(0)])
                sxs = []
                for h in range(H):
                    asrc = plsc.load_gather(recb[k], [rows, _i16(CA + h)])
                    adv = plsc.load_gather(drecb[k], [rows, _i16(h)])
                    z = adv + asrc + ea16 * ch[h]
                    z = jnp.maximum(z, 0.2 * z)
                    sxs.append(jnp.exp(z))
                for h in range(H):
                    sx = sxs[h]
                    hvs = [plsc.load_gather(recb[k], [rows, _i16(h * D + d)])
                           for d in range(D)]
                    for d in range(D):
                        plsc.store_scatter(outb[k], [rows, _i16(h * D + d)],
                                           sx * hvs[d])
                    plsc.store_scatter(outb[k], [rows, _i16(H * D + h)], sx)
                    plsc.store_scatter(outb[k], [rows, _i16(H * D + H + h)],
                                       sx * ea16)
            plsc.parallel_loop(0, C, 16, unroll=2)(grp)

        issue_gathers(0, 0)

        def pipe(t, carry):
            c0 = 2 * t
            issue_gathers(c0 + 1, 1)
            wait_gathers(c0, 0)

            @pl.when(t > 0)
            def _():
                wait_scatter(c0 - 2, 0)
            compute(c0, 0)
            issue_scatter(c0, 0)
            issue_gathers(c0 + 2, 0)
            wait_gathers(c0 + 1, 1)

            @pl.when(t > 0)
            def _():
                wait_scatter(c0 - 1, 1)
            compute(c0 + 1, 1)
            issue_scatter(c0 + 1, 1)
            return carry
        lax.fori_loop(0, (CHUNKS - 1) // 2, pipe, 0)

        last = CHUNKS - 1
        wait_gathers(last, 0)
        wait_scatter(last - 2, 0)
        compute(last, 0)
        issue_scatter(last, 0)
        wait_scatter(last - 1, 1)
        wait_scatter(last, 0)

        plsc.subcore_barrier()
        pltpu.sync_copy(accum.at[pl.ds(si * RPT, RPT)],
                        out_r.at[ci].at[pl.ds(si * RPT, RPT)])

    return body(table, adst, edge_index, ea, cvec, zeros)


def _sc_edge_pass2(table, edge_index, ea, cvec, zeros):
    D, WA = 7, 16
    mesh = plsc.VectorSubcoreMesh(core_axis_name="c", subcore_axis_name="s",
                                  num_cores=NC, num_subcores=NS)

    @functools.partial(
        pl.kernel,
        out_type=jax.ShapeDtypeStruct((NC, N, WA), jnp.float32),
        mesh=mesh,
        compiler_params=pltpu.CompilerParams(use_tc_tiling_on_sc=False,
                                             needs_layout_passes=False),
        scratch_types=[
            pltpu.VMEM((PT,), jnp.int32),
            pltpu.VMEM((PT,), jnp.int32),
            [pltpu.VMEM((C, 16), jnp.float32)] * 2,
            [pltpu.VMEM((C, 16), jnp.float32)] * 2,
            [pltpu.VMEM((C, 1), jnp.float32)] * 2,
            [pltpu.VMEM((C, WA), jnp.float32)] * 2,
            pltpu.VMEM((16, 16), jnp.float32),
            pltpu.VMEM_SHARED((N, WA), jnp.float32),
            [pltpu.SemaphoreType.DMA] * 2,
            [pltpu.SemaphoreType.DMA] * 2,
        ],
    )
    def body(table_r, edge_r, ea_r, cvec_r, zeros_r, out_r,
             srcv, dstv, recb, drecb, eab, outb, cb, accum, sg, ss):
        ci = lax.axis_index("c")
        si = lax.axis_index("s")
        wid = si * NC + ci
        g0 = wid * PT
        iota = lax.iota(jnp.int32, 16)

        pltpu.sync_copy(edge_r.at[0].at[pl.ds(g0, PT)], srcv)
        pltpu.sync_copy(edge_r.at[1].at[pl.ds(g0, PT)], dstv)
        pltpu.sync_copy(cvec_r, cb)
        pltpu.sync_copy(zeros_r.at[pl.ds(si * RPT, RPT)],
                        accum.at[pl.ds(si * RPT, RPT)])

        def zrow(g, carry):
            for k in range(2):
                plsc.store_scatter(outb[k], [_i16(0) + g, iota], _f16(0.0))
            return carry
        lax.fori_loop(0, C, zrow, 0)

        c2 = plsc.load_gather(cb, [_i16(0), iota])
        we2 = plsc.load_gather(cb, [_i16(1), iota])
        plsc.subcore_barrier()

        def issue_ea(i, k):
            pltpu.async_copy(table_r.at[srcv.at[pl.ds(i * C, C)]],
                             recb[k], sg[k])
            pltpu.async_copy(table_r.at[dstv.at[pl.ds(i * C, C)]],
                             drecb[k], sg[k])
            pltpu.async_copy(ea_r.at[pl.ds(g0 + i * C, C)], eab[k], sg[k])

        def wait_ea(i, k):
            pltpu.make_async_copy(table_r.at[srcv.at[pl.ds(i * C, C)]],
                                  recb[k], sg[k]).wait()
            pltpu.make_async_copy(table_r.at[dstv.at[pl.ds(i * C, C)]],
                                  drecb[k], sg[k]).wait()
            pltpu.make_async_copy(ea_r.at[pl.ds(g0 + i * C, C)],
                                  eab[k], sg[k]).wait()

        def issue_scatter(i, k):
            pltpu.async_copy(outb[k], accum.at[dstv.at[pl.ds(i * C, C)]],
                             ss[k], add=True)

        def wait_scatter(i, k):
            pltpu.make_async_copy(outb[k], accum.at[dstv.at[pl.ds(i * C, C)]],
                                  ss[k]).wait()

        def compute(i, k):
            ibase = _i16(0) + i * C

            def grp(jb):
                rows = jb + iota
                ea16 = plsc.load_gather(eab[k], [rows, _i16(0)])
                asrc = plsc.load_gather(recb[k], [rows, _i16(7)])
                adv = plsc.load_gather(drecb[k], [rows, _i16(8)])
                z = adv + asrc + ea16 * c2
                z = jnp.maximum(z, 0.2 * z)
                sx = jnp.exp(z)
                hvs = [plsc.load_gather(recb[k], [rows, _i16(d)])
                       for d in range(D)]
                for d in range(D):
                    plsc.store_scatter(outb[k], [rows, _i16(d)], sx * hvs[d])
                plsc.store_scatter(outb[k], [rows, _i16(D)], sx)
                plsc.store_scatter(outb[k], [rows, _i16(D + 1)],
                                   sx * ea16 * we2)
            plsc.parallel_loop(0, C, 16, unroll=2)(grp)

        issue_ea(0, 0)

        def pipe(t, carry):
            c0 = 2 * t
            issue_ea(c0 + 1, 1)
            wait_ea(c0, 0)

            @pl.when(t > 0)
            def _():
                wait_scatter(c0 - 2, 0)
            compute(c0, 0)
            issue_scatter(c0, 0)
            issue_ea(c0 + 2, 0)
            wait_ea(c0 + 1, 1)

            @pl.when(t > 0)
            def _():
                wait_scatter(c0 - 1, 1)
            compute(c0 + 1, 1)
            issue_scatter(c0 + 1, 1)
            return carry
        lax.fori_loop(0, (CHUNKS - 1) // 2, pipe, 0)

        last = CHUNKS - 1
        wait_ea(last, 0)
        wait_scatter(last - 2, 0)
        compute(last, 0)
        issue_scatter(last, 0)
        wait_scatter(last - 1, 1)
        wait_scatter(last, 0)

        plsc.subcore_barrier()
        pltpu.sync_copy(accum.at[pl.ds(si * RPT, RPT)],
                        out_r.at[ci].at[pl.ds(si * RPT, RPT)])

    return body(table, edge_index, ea, cvec, zeros)


def _tc_table1(x, W1r, vsrc, vdst):
    R = 2000

    def body(x_r, w_r, vs_r, vd_r, m8_r, t_r, a_r):
        m8 = m8_r[...]
        t = jnp.dot(x_r[...], w_r[...], preferred_element_type=jnp.float32)
        asrc = jnp.dot(t * vs_r[...], m8, preferred_element_type=jnp.float32)
        adst = jnp.dot(t * vd_r[...], m8, preferred_element_type=jnp.float32)
        t_r[...] = jnp.concatenate([t, asrc, adst], axis=1)
        a_r[...] = jnp.concatenate([adst, jnp.zeros((R, 8), jnp.float32)],
                                   axis=1)

    return pl.pallas_call(
        body,
        grid=(N // R,),
        in_specs=[pl.BlockSpec((R, 128), lambda i: (i, 0)),
                  pl.BlockSpec((128, 64), lambda i: (0, 0)),
                  pl.BlockSpec((1, 64), lambda i: (0, 0)),
                  pl.BlockSpec((1, 64), lambda i: (0, 0)),
                  pl.BlockSpec((64, 8), lambda i: (0, 0))],
        out_specs=[pl.BlockSpec((R, 80), lambda i: (i, 0)),
                   pl.BlockSpec((R, 16), lambda i: (i, 0))],
        out_shape=[jax.ShapeDtypeStruct((N, 80), jnp.float32),
                   jax.ShapeDtypeStruct((N, 16), jnp.float32)],
    )(x, W1r, vsrc, vdst, jnp.asarray(_M8))


def _tc_table2(acc1, w32, W2_0, a2s, a2d):
    R = 2000

    def body(acc_r, w32_r, w2_r, a2s_r, a2d_r, k8_r, k4_r, pn_r, pe_r, t_r):
        k8, k4, pn, pe = k8_r[...], k4_r[...], pn_r[...], pe_r[...]
        acc = acc_r[0] + acc_r[1]
        sh = acc[:, 0:64]
        sv = acc[:, 64:72]
        se = acc[:, 72:80]
        invd = 1.0 / (sv + 1e-16)
        nodep = sh * jnp.dot(invd, k8, preferred_element_type=jnp.float32)
        edgep = (jnp.dot(se * invd, k4, preferred_element_type=jnp.float32)
                 * w32_r[...])
        out1 = (jnp.dot(nodep, pn, preferred_element_type=jnp.float32)
                + jnp.dot(edgep, pe, preferred_element_type=jnp.float32))
        t2 = jnp.where(out1 > 0, out1, jnp.exp(out1) - 1.0)
        hh = jnp.dot(t2, w2_r[...], preferred_element_type=jnp.float32)
        c7 = jnp.dot(hh, a2s_r[...], preferred_element_type=jnp.float32)
        c8 = jnp.dot(hh, a2d_r[...], preferred_element_type=jnp.float32)
        t_r[...] = jnp.concatenate(
            [hh, c7, c8, jnp.zeros((R, 7), jnp.float32)], axis=1)

    return pl.pallas_call(
        body,
        grid=(N // R,),
        in_specs=[pl.BlockSpec((2, R, 80), lambda i: (0, i, 0)),
                  pl.BlockSpec((1, 32), lambda i: (0, 0)),
                  pl.BlockSpec((96, 7), lambda i: (0, 0)),
                  pl.BlockSpec((7, 1), lambda i: (0, 0)),
                  pl.BlockSpec((7, 1), lambda i: (0, 0)),
                  pl.BlockSpec((8, 64), lambda i: (0, 0)),
                  pl.BlockSpec((8, 32), lambda i: (0, 0)),
                  pl.BlockSpec((64, 96), lambda i: (0, 0)),
                  pl.BlockSpec((32, 96), lambda i: (0, 0))],
        out_specs=pl.BlockSpec((R, 16), lambda i: (i, 0)),
        out_shape=jax.ShapeDtypeStruct((N, 16), jnp.float32),
    )(acc1, w32, W2_0, a2s, a2d, jnp.asarray(_K8),
      jnp.asarray(_K4), jnp.asarray(_PN), jnp.asarray(_PE))


def _tc_final(acc2):
    R = 2000

    def body(acc_r, o_r):
        acc = acc_r[0] + acc_r[1]
        invd = 1.0 / (acc[:, 7:8] + 1e-16)
        out2 = jnp.concatenate(
            [acc[:, 0:7] * invd, acc[:, 8:9] * invd], axis=1)
        m = jnp.max(out2, axis=1, keepdims=True)
        lse = jnp.log(jnp.sum(jnp.exp(out2 - m), axis=1, keepdims=True))
        o_r[...] = out2 - m - lse

    return pl.pallas_call(
        body,
        grid=(N // R,),
        in_specs=[pl.BlockSpec((2, R, 16), lambda i: (0, i, 0))],
        out_specs=pl.BlockSpec((R, 8), lambda i: (i, 0)),
        out_shape=jax.ShapeDtypeStruct((N, 8), jnp.float32),
    )(acc2)


def kernel(x, edge_index, edge_attr, W1, We1, a1, W2, We2, a2):
    H = 8
    ei = edge_index.astype(jnp.int32)

    W1r = jnp.transpose(W1, (1, 0, 2)).reshape(128, 64)
    vsrc = a1[:, 8:16].reshape(1, 64)
    vdst = a1[:, 0:8].reshape(1, 64)
    c1 = jnp.einsum("ho,ho->h", We1[:, 0, :], a1[:, 16:20])
    cvec1 = jnp.zeros((16, 16), jnp.float32)
    cvec1 = cvec1.at[0:H, :].set(jnp.broadcast_to(c1[:, None], (H, 16)))

    w32 = We1[:, 0, :].reshape(1, 32)
    c2 = We2[0, 0, 0] * a2[0, 14]
    we2 = We2[0, 0, 0]
    cvec2 = jnp.zeros((16, 16), jnp.float32)
    cvec2 = cvec2.at[0, :].set(c2)
    cvec2 = cvec2.at[1, :].set(we2)

    z80 = jnp.zeros((N, 80), jnp.float32)
    z16 = jnp.zeros((N, 16), jnp.float32)

    table1, adst1 = _tc_table1(x, W1r, vsrc, vdst)
    acc1 = _sc_edge_pass1(table1, adst1, ei, edge_attr, cvec1, z80)
    table2 = _tc_table2(acc1, w32, W2[0], a2[0, 7:14].reshape(7, 1),
                        a2[0, 0:7].reshape(7, 1))
    acc2 = _sc_edge_pass2(table2, ei, edge_attr, cvec2, z16)
    return _tc_final(acc2)


# speedup vs baseline: 1.7910x; 1.7910x over previous
import functools

import jax
import jax.numpy as jnp
import numpy as np
from jax import lax
from jax.experimental import pallas as pl
from jax.experimental.pallas import tpu as pltpu
from jax.experimental.pallas import tpu_sc as plsc

N = 10000
E = 320000
NC, NS = 2, 16
NW = NC * NS
PT = E // NW
C = 80
CHUNKS = PT // C
RPT = N // NS

_EYE8 = np.eye(8, dtype=np.float32)
_K8 = np.kron(_EYE8, np.ones((1, 8), np.float32))
_K4 = np.kron(_EYE8, np.ones((1, 4), np.float32))
_SELN = np.concatenate([np.eye(8), np.zeros((4, 8))], 0)
_PN = np.kron(_EYE8, _SELN.T.astype(np.float32))
_SELE = np.concatenate([np.zeros((8, 4)), np.eye(4)], 0)
_PE = np.kron(_EYE8, _SELE.T.astype(np.float32))
_M8 = np.kron(_EYE8, np.ones((8, 1), np.float32))


def _f16(v):
    return jnp.full((16,), v, dtype=jnp.float32)


def _i16(v):
    return jnp.full((16,), v, dtype=jnp.int32)


def _sc_edge_pass1(table, adst, edge_index, ea, cvec, zeros):
    H, D, CA, WT, WA = 8, 8, 64, 80, 80
    mesh = plsc.VectorSubcoreMesh(core_axis_name="c", subcore_axis_name="s",
                                  num_cores=NC, num_subcores=NS)

    @functools.partial(
        pl.kernel,
        out_type=jax.ShapeDtypeStruct((NC, N, WA), jnp.float32),
        mesh=mesh,
        compiler_params=pltpu.CompilerParams(use_tc_tiling_on_sc=False,
                                             needs_layout_passes=False),
        scratch_types=[
            pltpu.VMEM((PT,), jnp.int32),
            pltpu.VMEM((PT,), jnp.int32),
            pltpu.VMEM((CHUNKS, C), jnp.float32),
            [pltpu.VMEM((C, WT), jnp.float32)] * 2,
            [pltpu.VMEM((C, 16), jnp.float32)] * 2,
            [pltpu.VMEM((C, WA), jnp.float32)] * 2,
            pltpu.VMEM((16, 16), jnp.float32),
            pltpu.VMEM_SHARED((N, WA), jnp.float32),
            [pltpu.SemaphoreType.DMA] * 2,
            [pltpu.SemaphoreType.DMA] * 2,
        ],
    )
    def body(table_r, adst_r, edge_r, ea_r, cvec_r, zeros_r, out_r,
             srcv, dstv, eav, recb, drecb, outb, cb, accum, sg, ss):
        ci = lax.axis_index("c")
        si = lax.axis_index("s")
        wid = si * NC + ci
        g0 = wid * PT
        iota = lax.iota(jnp.int32, 16)

        pltpu.sync_copy(edge_r.at[0].at[pl.ds(g0, PT)], srcv)
        pltpu.sync_copy(edge_r.at[1].at[pl.ds(g0, PT)], dstv)
        pltpu.sync_copy(ea_r.at[pl.ds(wid * CHUNKS, CHUNKS)], eav)
        pltpu.sync_copy(cvec_r, cb)
        pltpu.sync_copy(zeros_r.at[pl.ds(si * RPT, RPT)],
                        accum.at[pl.ds(si * RPT, RPT)])

        ch = [plsc.load_gather(cb, [_i16(h), iota]) for h in range(H)]
        plsc.subcore_barrier()

        def issue_gathers(i, k):
            pltpu.async_copy(table_r.at[srcv.at[pl.ds(i * C, C)]],
                             recb[k], sg[k])
            pltpu.async_copy(adst_r.at[dstv.at[pl.ds(i * C, C)]],
                             drecb[k], sg[k])

        def wait_gathers(i, k):
            pltpu.make_async_copy(table_r.at[srcv.at[pl.ds(i * C, C)]],
                                  recb[k], sg[k]).wait()
            pltpu.make_async_copy(adst_r.at[dstv.at[pl.ds(i * C, C)]],
                                  drecb[k], sg[k]).wait()

        def issue_scatter(i, k):
            pltpu.async_copy(outb[k], accum.at[dstv.at[pl.ds(i * C, C)]],
                             ss[k], add=True)

        def wait_scatter(i, k):
            pltpu.make_async_copy(outb[k], accum.at[dstv.at[pl.ds(i * C, C)]],
                                  ss[k]).wait()

        def compute(i, k):
            ibase = _i16(0) + i

            def grp(jb):
                rows = jb + iota
                ea16 = plsc.load_gather(eav, [ibase, rows])
                sxs = []
                for h in range(H):
                    asrc = plsc.load_gather(recb[k], [rows, _i16(CA + h)])
                    adv = plsc.load_gather(drecb[k], [rows, _i16(h)])
                    z = adv + asrc + ea16 * ch[h]
                    z = jnp.maximum(z, 0.2 * z)
                    sxs.append(jnp.exp(z))
                for h in range(H):
                    sx = sxs[h]
                    hvs = [plsc.load_gather(recb[k], [rows, _i16(h * D + d)])
                           for d in range(D)]
                    for d in range(D):
                        plsc.store_scatter(outb[k], [rows, _i16(h * D + d)],
                                           sx * hvs[d])
                    plsc.store_scatter(outb[k], [rows, _i16(H * D + h)], sx)
                    plsc.store_scatter(outb[k], [rows, _i16(H * D + H + h)],
                                       sx * ea16)
            plsc.parallel_loop(0, C, 16, unroll=2)(grp)

        issue_gathers(0, 0)

        def pipe(t, carry):
            c0 = 2 * t
            issue_gathers(c0 + 1, 1)
            wait_gathers(c0, 0)

            @pl.when(t > 0)
            def _():
                wait_scatter(c0 - 2, 0)
            compute(c0, 0)
            issue_scatter(c0, 0)
            issue_gathers(c0 + 2, 0)
            wait_gathers(c0 + 1, 1)

            @pl.when(t > 0)
            def _():
                wait_scatter(c0 - 1, 1)
            compute(c0 + 1, 1)
            issue_scatter(c0 + 1, 1)
            return carry
        lax.fori_loop(0, (CHUNKS - 1) // 2, pipe, 0)

        last = CHUNKS - 1
        wait_gathers(last, 0)
        wait_scatter(last - 2, 0)
        compute(last, 0)
        issue_scatter(last, 0)
        wait_scatter(last - 1, 1)
        wait_scatter(last, 0)

        plsc.subcore_barrier()
        pltpu.sync_copy(accum.at[pl.ds(si * RPT, RPT)],
                        out_r.at[ci].at[pl.ds(si * RPT, RPT)])

    return body(table, adst, edge_index, ea, cvec, zeros)


def _sc_edge_pass2(table, edge_index, ea, cvec, zeros):
    D, WA = 7, 16
    mesh = plsc.VectorSubcoreMesh(core_axis_name="c", subcore_axis_name="s",
                                  num_cores=NC, num_subcores=NS)

    @functools.partial(
        pl.kernel,
        out_type=jax.ShapeDtypeStruct((NC, N, WA), jnp.float32),
        mesh=mesh,
        compiler_params=pltpu.CompilerParams(use_tc_tiling_on_sc=False,
                                             needs_layout_passes=False),
        scratch_types=[
            pltpu.VMEM((PT,), jnp.int32),
            pltpu.VMEM((PT,), jnp.int32),
            pltpu.VMEM((CHUNKS, C), jnp.float32),
            [pltpu.VMEM((C, 16), jnp.float32)] * 2,
            [pltpu.VMEM((C, 16), jnp.float32)] * 2,
            [pltpu.VMEM((C, WA), jnp.float32)] * 2,
            pltpu.VMEM((16, 16), jnp.float32),
            pltpu.VMEM_SHARED((N, WA), jnp.float32),
            [pltpu.SemaphoreType.DMA] * 2,
            [pltpu.SemaphoreType.DMA] * 2,
        ],
    )
    def body(table_r, edge_r, ea_r, cvec_r, zeros_r, out_r,
             srcv, dstv, eav, recb, drecb, outb, cb, accum, sg, ss):
        ci = lax.axis_index("c")
        si = lax.axis_index("s")
        wid = si * NC + ci
        g0 = wid * PT
        iota = lax.iota(jnp.int32, 16)

        pltpu.sync_copy(edge_r.at[0].at[pl.ds(g0, PT)], srcv)
        pltpu.sync_copy(edge_r.at[1].at[pl.ds(g0, PT)], dstv)
        pltpu.sync_copy(ea_r.at[pl.ds(wid * CHUNKS, CHUNKS)], eav)
        pltpu.sync_copy(cvec_r, cb)
        pltpu.sync_copy(zeros_r.at[pl.ds(si * RPT, RPT)],
                        accum.at[pl.ds(si * RPT, RPT)])

        def zrow(g, carry):
            for k in range(2):
                plsc.store_scatter(outb[k], [_i16(0) + g, iota], _f16(0.0))
            return carry
        lax.fori_loop(0, C, zrow, 0)

        c2 = plsc.load_gather(cb, [_i16(0), iota])
        we2 = plsc.load_gather(cb, [_i16(1), iota])
        plsc.subcore_barrier()

        def issue_ea(i, k):
            pltpu.async_copy(table_r.at[srcv.at[pl.ds(i * C, C)]],
                             recb[k], sg[k])
            pltpu.async_copy(table_r.at[dstv.at[pl.ds(i * C, C)]],
                             drecb[k], sg[k])

        def wait_ea(i, k):
            pltpu.make_async_copy(table_r.at[srcv.at[pl.ds(i * C, C)]],
                                  recb[k], sg[k]).wait()
            pltpu.make_async_copy(table_r.at[dstv.at[pl.ds(i * C, C)]],
                                  drecb[k], sg[k]).wait()

        def issue_scatter(i, k):
            pltpu.async_copy(outb[k], accum.at[dstv.at[pl.ds(i * C, C)]],
                             ss[k], add=True)

        def wait_scatter(i, k):
            pltpu.make_async_copy(outb[k], accum.at[dstv.at[pl.ds(i * C, C)]],
                                  ss[k]).wait()

        def compute(i, k):
            ibase = _i16(0) + i

            def grp(jb):
                rows = jb + iota
                ea16 = plsc.load_gather(eav, [ibase, rows])
                asrc = plsc.load_gather(recb[k], [rows, _i16(7)])
                adv = plsc.load_gather(drecb[k], [rows, _i16(8)])
                z = adv + asrc + ea16 * c2
                z = jnp.maximum(z, 0.2 * z)
                sx = jnp.exp(z)
                hvs = [plsc.load_gather(recb[k], [rows, _i16(d)])
                       for d in range(D)]
                for d in range(D):
                    plsc.store_scatter(outb[k], [rows, _i16(d)], sx * hvs[d])
                plsc.store_scatter(outb[k], [rows, _i16(D)], sx)
                plsc.store_scatter(outb[k], [rows, _i16(D + 1)],
                                   sx * ea16 * we2)
            plsc.parallel_loop(0, C, 16, unroll=2)(grp)

        issue_ea(0, 0)

        def pipe(t, carry):
            c0 = 2 * t
            issue_ea(c0 + 1, 1)
            wait_ea(c0, 0)

            @pl.when(t > 0)
            def _():
                wait_scatter(c0 - 2, 0)
            compute(c0, 0)
            issue_scatter(c0, 0)
            issue_ea(c0 + 2, 0)
            wait_ea(c0 + 1, 1)

            @pl.when(t > 0)
            def _():
                wait_scatter(c0 - 1, 1)
            compute(c0 + 1, 1)
            issue_scatter(c0 + 1, 1)
            return carry
        lax.fori_loop(0, (CHUNKS - 1) // 2, pipe, 0)

        last = CHUNKS - 1
        wait_ea(last, 0)
        wait_scatter(last - 2, 0)
        compute(last, 0)
        issue_scatter(last, 0)
        wait_scatter(last - 1, 1)
        wait_scatter(last, 0)

        plsc.subcore_barrier()
        pltpu.sync_copy(accum.at[pl.ds(si * RPT, RPT)],
                        out_r.at[ci].at[pl.ds(si * RPT, RPT)])

    return body(table, edge_index, ea, cvec, zeros)


def _tc_table1(x, W1r, vsrc, vdst):
    R = 2000

    def body(x_r, w_r, vs_r, vd_r, m8_r, t_r, a_r):
        m8 = m8_r[...]
        t = jnp.dot(x_r[...], w_r[...], preferred_element_type=jnp.float32)
        asrc = jnp.dot(t * vs_r[...], m8, preferred_element_type=jnp.float32)
        adst = jnp.dot(t * vd_r[...], m8, preferred_element_type=jnp.float32)
        t_r[...] = jnp.concatenate([t, asrc, adst], axis=1)
        a_r[...] = jnp.concatenate([adst, jnp.zeros((R, 8), jnp.float32)],
                                   axis=1)

    return pl.pallas_call(
        body,
        grid=(N // R,),
        in_specs=[pl.BlockSpec((R, 128), lambda i: (i, 0)),
                  pl.BlockSpec((128, 64), lambda i: (0, 0)),
                  pl.BlockSpec((1, 64), lambda i: (0, 0)),
                  pl.BlockSpec((1, 64), lambda i: (0, 0)),
                  pl.BlockSpec((64, 8), lambda i: (0, 0))],
        out_specs=[pl.BlockSpec((R, 80), lambda i: (i, 0)),
                   pl.BlockSpec((R, 16), lambda i: (i, 0))],
        out_shape=[jax.ShapeDtypeStruct((N, 80), jnp.float32),
                   jax.ShapeDtypeStruct((N, 16), jnp.float32)],
    )(x, W1r, vsrc, vdst, jnp.asarray(_M8))


def _tc_table2(acc1, w32, W2_0, a2s, a2d):
    R = 2000

    def body(acc_r, w32_r, w2_r, a2s_r, a2d_r, k8_r, k4_r, pn_r, pe_r, t_r):
        k8, k4, pn, pe = k8_r[...], k4_r[...], pn_r[...], pe_r[...]
        acc = acc_r[0] + acc_r[1]
        sh = acc[:, 0:64]
        sv = acc[:, 64:72]
        se = acc[:, 72:80]
        invd = 1.0 / (sv + 1e-16)
        nodep = sh * jnp.dot(invd, k8, preferred_element_type=jnp.float32)
        edgep = (jnp.dot(se * invd, k4, preferred_element_type=jnp.float32)
                 * w32_r[...])
        out1 = (jnp.dot(nodep, pn, preferred_element_type=jnp.float32)
                + jnp.dot(edgep, pe, preferred_element_type=jnp.float32))
        t2 = jnp.where(out1 > 0, out1, jnp.exp(out1) - 1.0)
        hh = jnp.dot(t2, w2_r[...], preferred_element_type=jnp.float32)
        c7 = jnp.dot(hh, a2s_r[...], preferred_element_type=jnp.float32)
        c8 = jnp.dot(hh, a2d_r[...], preferred_element_type=jnp.float32)
        t_r[...] = jnp.concatenate(
            [hh, c7, c8, jnp.zeros((R, 7), jnp.float32)], axis=1)

    return pl.pallas_call(
        body,
        grid=(N // R,),
        in_specs=[pl.BlockSpec((2, R, 80), lambda i: (0, i, 0)),
                  pl.BlockSpec((1, 32), lambda i: (0, 0)),
                  pl.BlockSpec((96, 7), lambda i: (0, 0)),
                  pl.BlockSpec((7, 1), lambda i: (0, 0)),
                  pl.BlockSpec((7, 1), lambda i: (0, 0)),
                  pl.BlockSpec((8, 64), lambda i: (0, 0)),
                  pl.BlockSpec((8, 32), lambda i: (0, 0)),
                  pl.BlockSpec((64, 96), lambda i: (0, 0)),
                  pl.BlockSpec((32, 96), lambda i: (0, 0))],
        out_specs=pl.BlockSpec((R, 16), lambda i: (i, 0)),
        out_shape=jax.ShapeDtypeStruct((N, 16), jnp.float32),
    )(acc1, w32, W2_0, a2s, a2d, jnp.asarray(_K8),
      jnp.asarray(_K4), jnp.asarray(_PN), jnp.asarray(_PE))


def _tc_final(acc2):
    R = 2000

    def body(acc_r, o_r):
        acc = acc_r[0] + acc_r[1]
        invd = 1.0 / (acc[:, 7:8] + 1e-16)
        out2 = jnp.concatenate(
            [acc[:, 0:7] * invd, acc[:, 8:9] * invd], axis=1)
        m = jnp.max(out2, axis=1, keepdims=True)
        lse = jnp.log(jnp.sum(jnp.exp(out2 - m), axis=1, keepdims=True))
        o_r[...] = out2 - m - lse

    return pl.pallas_call(
        body,
        grid=(N // R,),
        in_specs=[pl.BlockSpec((2, R, 16), lambda i: (0, i, 0))],
        out_specs=pl.BlockSpec((R, 8), lambda i: (i, 0)),
        out_shape=jax.ShapeDtypeStruct((N, 8), jnp.float32),
    )(acc2)


def kernel(x, edge_index, edge_attr, W1, We1, a1, W2, We2, a2):
    H = 8
    ei = edge_index.astype(jnp.int32)

    W1r = jnp.transpose(W1, (1, 0, 2)).reshape(128, 64)
    vsrc = a1[:, 8:16].reshape(1, 64)
    vdst = a1[:, 0:8].reshape(1, 64)
    c1 = jnp.einsum("ho,ho->h", We1[:, 0, :], a1[:, 16:20])
    cvec1 = jnp.zeros((16, 16), jnp.float32)
    cvec1 = cvec1.at[0:H, :].set(jnp.broadcast_to(c1[:, None], (H, 16)))

    w32 = We1[:, 0, :].reshape(1, 32)
    c2 = We2[0, 0, 0] * a2[0, 14]
    we2 = We2[0, 0, 0]
    cvec2 = jnp.zeros((16, 16), jnp.float32)
    cvec2 = cvec2.at[0, :].set(c2)
    cvec2 = cvec2.at[1, :].set(we2)

    z80 = jnp.zeros((N, 80), jnp.float32)
    z16 = jnp.zeros((N, 16), jnp.float32)

    ea2d = edge_attr.reshape(E // C, C)

    table1, adst1 = _tc_table1(x, W1r, vsrc, vdst)
    acc1 = _sc_edge_pass1(table1, adst1, ei, ea2d, cvec1, z80)
    table2 = _tc_table2(acc1, w32, W2[0], a2[0, 7:14].reshape(7, 1),
                        a2[0, 0:7].reshape(7, 1))
    acc2 = _sc_edge_pass2(table2, ei, ea2d, cvec2, z16)
    return _tc_final(acc2)
